# Initial kernel scaffold; baseline (speedup 1.0000x reference)
import functools

import jax
import jax.numpy as jnp
from jax import lax
from jax.experimental import pallas as pl
from jax.experimental.pallas import tpu as pltpu
from jax.experimental.pallas import tpu_sc as plsc

NC = 2
NS = 16
SCATTER_B = 80
CHUNK = 2000


def _sc_scatter_partials(edge_attr, dest2d, zeros_nx16, ones_bx16, n_nodes):
    e, de = edge_attr.shape
    nw = NC * NS
    epw = e // nw
    n_chunks = epw // CHUNK
    n_batches = CHUNK // SCATTER_B
    rows_per_tile = n_nodes // NS

    mesh = plsc.VectorSubcoreMesh(
        core_axis_name="c", subcore_axis_name="s",
        num_cores=NC, num_subcores=NS)

    @functools.partial(
        pl.kernel,
        out_type=[
            jax.ShapeDtypeStruct((NC, n_nodes, de), jnp.float32),
            jax.ShapeDtypeStruct((NC, n_nodes, de), jnp.float32),
        ],
        mesh=mesh,
        scratch_types=[
            pltpu.VMEM_SHARED((n_nodes, de), jnp.float32),
            pltpu.VMEM_SHARED((n_nodes, de), jnp.float32),
            pltpu.VMEM((CHUNK, de), jnp.float32),
            pltpu.VMEM((n_batches, SCATTER_B), jnp.int32),
            pltpu.VMEM((SCATTER_B, de), jnp.float32),
        ],
    )
    def body(attr_hbm, idx_hbm, zeros_hbm, ones_hbm, psum, pcnt,
             sums_sh, cnt_sh, attr_v, idx_v, ones_v):
        c = lax.axis_index("c")
        s = lax.axis_index("s")
        w = c * NS + s

        rbase = s * rows_per_tile
        pltpu.sync_copy(zeros_hbm.at[pl.ds(rbase, rows_per_tile)],
                        sums_sh.at[pl.ds(rbase, rows_per_tile)])
        pltpu.sync_copy(zeros_hbm.at[pl.ds(rbase, rows_per_tile)],
                        cnt_sh.at[pl.ds(rbase, rows_per_tile)])
        pltpu.sync_copy(ones_hbm, ones_v)
        plsc.subcore_barrier()

        ebase = w * epw
        bbase = w * (epw // SCATTER_B)
        for k in range(n_chunks):
            pltpu.sync_copy(attr_hbm.at[pl.ds(ebase + k * CHUNK, CHUNK)],
                            attr_v)
            pltpu.sync_copy(idx_hbm.at[pl.ds(bbase + k * n_batches,
                                             n_batches)], idx_v)

            def scat(j, carry):
                idx = idx_v.at[j]
                pltpu.sync_copy(attr_v.at[pl.ds(j * SCATTER_B, SCATTER_B)],
                                sums_sh.at[idx], add=True)
                pltpu.sync_copy(ones_v, cnt_sh.at[idx], add=True)
                return carry

            lax.fori_loop(0, n_batches, scat, 0)

        plsc.subcore_barrier()

        pltpu.sync_copy(sums_sh.at[pl.ds(rbase, rows_per_tile)],
                        psum.at[c, pl.ds(rbase, rows_per_tile)])
        pltpu.sync_copy(cnt_sh.at[pl.ds(rbase, rows_per_tile)],
                        pcnt.at[c, pl.ds(rbase, rows_per_tile)])

    return body(edge_attr, dest2d, zeros_nx16, ones_bx16)


def _tc_mlp_kernel(x_ref, psum_ref, pcnt_ref, batch_ref, u_ref,
                   w1_ref, b1_ref, w2_ref, b2_ref, out_ref, *, d, de, g):
    sums = psum_ref[0] + psum_ref[1]
    cnt = pcnt_ref[0, :, 0:1] + pcnt_ref[1, :, 0:1]
    agg = sums / jnp.maximum(cnt, 1.0)

    onehot = (batch_ref[...] ==
              lax.broadcasted_iota(jnp.int32, (1, g), 1)).astype(jnp.float32)
    uw = jnp.dot(u_ref[...], w1_ref[d + de:, :],
                 preferred_element_type=jnp.float32)
    h = jnp.dot(x_ref[...], w1_ref[:d, :],
                preferred_element_type=jnp.float32)
    h += jnp.dot(agg, w1_ref[d:d + de, :],
                 preferred_element_type=jnp.float32)
    h += jnp.dot(onehot, uw, preferred_element_type=jnp.float32)
    h = jnp.maximum(h + b1_ref[...], 0.0)
    out = jnp.dot(h, w2_ref[...], preferred_element_type=jnp.float32)
    out_ref[...] = jnp.maximum(out + b2_ref[...], 0.0)


def kernel(x, edge_index, edge_attr, u, batch, W1, b1, W2, b2):
    n, d = x.shape
    e, de = edge_attr.shape
    g = u.shape[0]
    lat = W2.shape[1]

    dest2d = edge_index[1].reshape(e // SCATTER_B, SCATTER_B)
    zeros_nx16 = jnp.zeros((n, de), dtype=jnp.float32)
    ones_bx16 = jnp.ones((SCATTER_B, de), dtype=jnp.float32)

    psum, pcnt = _sc_scatter_partials(edge_attr, dest2d, zeros_nx16,
                                      ones_bx16, n)

    bn = 1000
    grid = n // bn
    tc = pl.pallas_call(
        functools.partial(_tc_mlp_kernel, d=d, de=de, g=g),
        grid=(grid,),
        in_specs=[
            pl.BlockSpec((bn, d), lambda i: (i, 0)),
            pl.BlockSpec((NC, bn, de), lambda i: (0, i, 0)),
            pl.BlockSpec((NC, bn, de), lambda i: (0, i, 0)),
            pl.BlockSpec((bn, 1), lambda i: (i, 0)),
            pl.BlockSpec((g, d), lambda i: (0, 0)),
            pl.BlockSpec(W1.shape, lambda i: (0, 0)),
            pl.BlockSpec((1, lat), lambda i: (0, 0)),
            pl.BlockSpec(W2.shape, lambda i: (0, 0)),
            pl.BlockSpec((1, lat), lambda i: (0, 0)),
        ],
        out_specs=pl.BlockSpec((bn, lat), lambda i: (i, 0)),
        out_shape=jax.ShapeDtypeStruct((n, lat), jnp.float32),
    )
    return tc(x, psum, pcnt, batch.reshape(n, 1), u,
              W1, b1.reshape(1, lat), W2, b2.reshape(1, lat))


# speedup vs baseline: 6.0492x; 6.0492x over previous
import functools

import jax
import jax.numpy as jnp
from jax import lax
from jax.experimental import pallas as pl
from jax.experimental.pallas import tpu as pltpu
from jax.experimental.pallas import tpu_sc as plsc

NC = 2
NS = 16
SCATTER_B = 80
CHUNK = 2000


def _sc_scatter_partials(edge_attr, dest3d, zeros_init, ones_bx16, n_pad):
    e, de = edge_attr.shape
    nw = NC * NS
    epw = e // nw
    n_chunks = epw // CHUNK
    n_batches = CHUNK // SCATTER_B
    batches_pw = epw // SCATTER_B
    rows_per_tile = n_pad // NS

    mesh = plsc.VectorSubcoreMesh(
        core_axis_name="c", subcore_axis_name="s",
        num_cores=NC, num_subcores=NS)

    @functools.partial(
        pl.kernel,
        out_type=[
            jax.ShapeDtypeStruct((NC, n_pad, de), jnp.float32),
            jax.ShapeDtypeStruct((NC, n_pad, de), jnp.float32),
        ],
        mesh=mesh,
        compiler_params=pltpu.CompilerParams(use_tc_tiling_on_sc=False),
        scratch_types=[
            pltpu.VMEM_SHARED((n_pad, de), jnp.float32),
            pltpu.VMEM_SHARED((n_pad, de), jnp.float32),
            pltpu.VMEM((CHUNK, de), jnp.float32),
            pltpu.VMEM((batches_pw, SCATTER_B), jnp.int32),
            pltpu.VMEM((SCATTER_B, de), jnp.float32),
        ],
    )
    def body(attr_hbm, idx_hbm, zeros_hbm, ones_hbm, psum, pcnt,
             sums_sh, cnt_sh, attr_v, idx_v, ones_v):
        c = lax.axis_index("c")
        s = lax.axis_index("s")
        w = c * NS + s

        rbase = s * rows_per_tile
        pltpu.sync_copy(zeros_hbm,
                        sums_sh.at[pl.ds(rbase, rows_per_tile)])
        pltpu.sync_copy(zeros_hbm,
                        cnt_sh.at[pl.ds(rbase, rows_per_tile)])
        pltpu.sync_copy(ones_hbm, ones_v)
        pltpu.sync_copy(idx_hbm.at[w], idx_v)
        plsc.subcore_barrier()

        ebase = w * epw
        for k in range(n_chunks):
            pltpu.sync_copy(attr_hbm.at[pl.ds(ebase + k * CHUNK, CHUNK)],
                            attr_v)

            def scat(j, carry):
                idx = idx_v.at[k * n_batches + j]
                pltpu.sync_copy(attr_v.at[pl.ds(j * SCATTER_B, SCATTER_B)],
                                sums_sh.at[idx], add=True)
                pltpu.sync_copy(ones_v, cnt_sh.at[idx], add=True)
                return carry

            lax.fori_loop(0, n_batches, scat, 0)

        plsc.subcore_barrier()

        pltpu.sync_copy(sums_sh.at[pl.ds(rbase, rows_per_tile)],
                        psum.at[c, pl.ds(rbase, rows_per_tile)])
        pltpu.sync_copy(cnt_sh.at[pl.ds(rbase, rows_per_tile)],
                        pcnt.at[c, pl.ds(rbase, rows_per_tile)])

    return body(edge_attr, dest3d, zeros_init, ones_bx16)


def _tc_mlp_kernel(x_ref, psum_ref, pcnt_ref, batch_ref, u_ref,
                   w1_ref, b1_ref, w2_ref, b2_ref, out_ref, *, d, de, g):
    sums = psum_ref[0] + psum_ref[1]
    cnt = pcnt_ref[0, :, 0:1] + pcnt_ref[1, :, 0:1]
    agg = sums / jnp.maximum(cnt, 1.0)

    onehot = (batch_ref[...] ==
              lax.broadcasted_iota(jnp.int32, (1, g), 1)).astype(jnp.float32)
    uw = jnp.dot(u_ref[...], w1_ref[d + de:, :],
                 preferred_element_type=jnp.float32)
    h = jnp.dot(x_ref[...], w1_ref[:d, :],
                preferred_element_type=jnp.float32)
    h += jnp.dot(agg, w1_ref[d:d + de, :],
                 preferred_element_type=jnp.float32)
    h += jnp.dot(onehot, uw, preferred_element_type=jnp.float32)
    h = jnp.maximum(h + b1_ref[...], 0.0)
    out = jnp.dot(h, w2_ref[...], preferred_element_type=jnp.float32)
    out_ref[...] = jnp.maximum(out + b2_ref[...], 0.0)


def kernel(x, edge_index, edge_attr, u, batch, W1, b1, W2, b2):
    n, d = x.shape
    e, de = edge_attr.shape
    g = u.shape[0]
    lat = W2.shape[1]

    nw = NC * NS
    epw = e // nw
    n_pad = -(-n // (8 * NS)) * (8 * NS)
    dest3d = edge_index[1].reshape(nw, epw // SCATTER_B, SCATTER_B)
    zeros_init = jnp.zeros((n_pad // NS, de), dtype=jnp.float32)
    ones_bx16 = jnp.ones((SCATTER_B, de), dtype=jnp.float32)

    psum, pcnt = _sc_scatter_partials(edge_attr, dest3d, zeros_init,
                                      ones_bx16, n_pad)

    bn = 1000
    grid = n // bn
    tc = pl.pallas_call(
        functools.partial(_tc_mlp_kernel, d=d, de=de, g=g),
        grid=(grid,),
        in_specs=[
            pl.BlockSpec((bn, d), lambda i: (i, 0)),
            pl.BlockSpec((NC, bn, de), lambda i: (0, i, 0)),
            pl.BlockSpec((NC, bn, de), lambda i: (0, i, 0)),
            pl.BlockSpec((bn, 1), lambda i: (i, 0)),
            pl.BlockSpec((g, d), lambda i: (0, 0)),
            pl.BlockSpec(W1.shape, lambda i: (0, 0)),
            pl.BlockSpec((1, lat), lambda i: (0, 0)),
            pl.BlockSpec(W2.shape, lambda i: (0, 0)),
            pl.BlockSpec((1, lat), lambda i: (0, 0)),
        ],
        out_specs=pl.BlockSpec((bn, lat), lambda i: (i, 0)),
        out_shape=jax.ShapeDtypeStruct((n, lat), jnp.float32),
    )
    return tc(x, psum, pcnt, batch.reshape(n, 1), u,
              W1, b1.reshape(1, lat), W2, b2.reshape(1, lat))


# speedup vs baseline: 6.3969x; 1.0575x over previous
import functools

import jax
import jax.numpy as jnp
from jax import lax
from jax.experimental import pallas as pl
from jax.experimental.pallas import tpu as pltpu
from jax.experimental.pallas import tpu_sc as plsc

NC = 2
NS = 16
SCATTER_B = 80
CHUNK = 2000


def _sc_scatter_partials(edge_attr, dest3d, zeros_init, ones_bx16, n_pad):
    e, de = edge_attr.shape
    nw = NC * NS
    epw = e // nw
    n_chunks = epw // CHUNK
    n_batches = CHUNK // SCATTER_B
    batches_pw = epw // SCATTER_B
    rows_per_tile = n_pad // NS

    mesh = plsc.VectorSubcoreMesh(
        core_axis_name="c", subcore_axis_name="s",
        num_cores=NC, num_subcores=NS)

    @functools.partial(
        pl.kernel,
        out_type=[
            jax.ShapeDtypeStruct((NC, n_pad, de), jnp.float32),
            jax.ShapeDtypeStruct((NC, n_pad, de), jnp.float32),
        ],
        mesh=mesh,
        compiler_params=pltpu.CompilerParams(use_tc_tiling_on_sc=False),
        scratch_types=[
            pltpu.VMEM_SHARED((n_pad, de), jnp.float32),
            pltpu.VMEM_SHARED((n_pad, de), jnp.float32),
            pltpu.VMEM((CHUNK, de), jnp.float32),
            pltpu.VMEM((n_chunks, CHUNK), jnp.int32),
            pltpu.VMEM((CHUNK, de), jnp.float32),
        ],
    )
    def body(attr_hbm, idx_hbm, zeros_hbm, ones_hbm, psum, pcnt,
             sums_sh, cnt_sh, attr_v, idx_v, ones_v):
        c = lax.axis_index("c")
        s = lax.axis_index("s")
        w = c * NS + s

        rbase = s * rows_per_tile
        pltpu.sync_copy(zeros_hbm,
                        sums_sh.at[pl.ds(rbase, rows_per_tile)])
        pltpu.sync_copy(zeros_hbm,
                        cnt_sh.at[pl.ds(rbase, rows_per_tile)])
        pltpu.sync_copy(ones_hbm, ones_v)
        pltpu.sync_copy(idx_hbm.at[w], idx_v)
        plsc.subcore_barrier()

        ebase = w * epw
        for k in range(n_chunks):
            pltpu.sync_copy(attr_hbm.at[pl.ds(ebase + k * CHUNK, CHUNK)],
                            attr_v)
            idx = idx_v.at[k]
            pltpu.sync_copy(attr_v, sums_sh.at[idx], add=True)
            pltpu.sync_copy(ones_v, cnt_sh.at[idx], add=True)

        plsc.subcore_barrier()

        pltpu.sync_copy(sums_sh.at[pl.ds(rbase, rows_per_tile)],
                        psum.at[c, pl.ds(rbase, rows_per_tile)])
        pltpu.sync_copy(cnt_sh.at[pl.ds(rbase, rows_per_tile)],
                        pcnt.at[c, pl.ds(rbase, rows_per_tile)])

    return body(edge_attr, dest3d, zeros_init, ones_bx16)


def _tc_mlp_kernel(x_ref, psum_ref, pcnt_ref, batch_ref, u_ref,
                   w1_ref, b1_ref, w2_ref, b2_ref, out_ref, *, d, de, g):
    sums = psum_ref[0] + psum_ref[1]
    cnt = pcnt_ref[0, :, 0:1] + pcnt_ref[1, :, 0:1]
    agg = sums / jnp.maximum(cnt, 1.0)

    onehot = (batch_ref[...] ==
              lax.broadcasted_iota(jnp.int32, (1, g), 1)).astype(jnp.float32)
    uw = jnp.dot(u_ref[...], w1_ref[d + de:, :],
                 preferred_element_type=jnp.float32)
    h = jnp.dot(x_ref[...], w1_ref[:d, :],
                preferred_element_type=jnp.float32)
    h += jnp.dot(agg, w1_ref[d:d + de, :],
                 preferred_element_type=jnp.float32)
    h += jnp.dot(onehot, uw, preferred_element_type=jnp.float32)
    h = jnp.maximum(h + b1_ref[...], 0.0)
    out = jnp.dot(h, w2_ref[...], preferred_element_type=jnp.float32)
    out_ref[...] = jnp.maximum(out + b2_ref[...], 0.0)


def kernel(x, edge_index, edge_attr, u, batch, W1, b1, W2, b2):
    n, d = x.shape
    e, de = edge_attr.shape
    g = u.shape[0]
    lat = W2.shape[1]

    nw = NC * NS
    epw = e // nw
    n_pad = -(-n // (8 * NS)) * (8 * NS)
    dest3d = edge_index[1].reshape(nw, epw // CHUNK, CHUNK)
    zeros_init = jnp.zeros((n_pad // NS, de), dtype=jnp.float32)
    ones_bx16 = jnp.ones((CHUNK, de), dtype=jnp.float32)

    psum, pcnt = _sc_scatter_partials(edge_attr, dest3d, zeros_init,
                                      ones_bx16, n_pad)

    bn = 1000
    grid = n // bn
    tc = pl.pallas_call(
        functools.partial(_tc_mlp_kernel, d=d, de=de, g=g),
        grid=(grid,),
        in_specs=[
            pl.BlockSpec((bn, d), lambda i: (i, 0)),
            pl.BlockSpec((NC, bn, de), lambda i: (0, i, 0)),
            pl.BlockSpec((NC, bn, de), lambda i: (0, i, 0)),
            pl.BlockSpec((bn, 1), lambda i: (i, 0)),
            pl.BlockSpec((g, d), lambda i: (0, 0)),
            pl.BlockSpec(W1.shape, lambda i: (0, 0)),
            pl.BlockSpec((1, lat), lambda i: (0, 0)),
            pl.BlockSpec(W2.shape, lambda i: (0, 0)),
            pl.BlockSpec((1, lat), lambda i: (0, 0)),
        ],
        out_specs=pl.BlockSpec((bn, lat), lambda i: (i, 0)),
        out_shape=jax.ShapeDtypeStruct((n, lat), jnp.float32),
    )
    return tc(x, psum, pcnt, batch.reshape(n, 1), u,
              W1, b1.reshape(1, lat), W2, b2.reshape(1, lat))
